# baseline (speedup 1.0000x reference)
import functools

import jax
import jax.numpy as jnp
from jax import lax
from jax.experimental import pallas as pl
from jax.experimental.pallas import tpu as pltpu
from jax.experimental.pallas import tpu_sc as plsc

B = 16384
E = 32
NF = 64

_info = plsc.get_sparse_core_info()
_NC, _NS = _info.num_cores, _info.num_subcores
NW = _NC * _NS
BPW = B // NW

_mesh = plsc.VectorSubcoreMesh(core_axis_name="c", subcore_axis_name="s")


@functools.partial(
    pl.kernel,
    mesh=_mesh,
    compiler_params=pltpu.CompilerParams(use_tc_tiling_on_sc=False),
    out_type=[
        jax.ShapeDtypeStruct((B, E), jnp.float32),
        jax.ShapeDtypeStruct((B, E), jnp.float32),
    ],
    scratch_types=[
        pltpu.VMEM((BPW,), jnp.int32),
        pltpu.VMEM((BPW,), jnp.int32),
        pltpu.VMEM((BPW, E), jnp.float32),
        pltpu.VMEM((BPW, E), jnp.float32),
        pltpu.SemaphoreType.DMA,
        pltpu.SemaphoreType.DMA,
    ],
)
def _sc_gather2(uid_hbm, iid_hbm, utab_hbm, itab_hbm,
                uout_hbm, iout_hbm,
                uidx_v, iidx_v, urows_v, irows_v, usem, isem):
    wid = lax.axis_index("s") * _NC + lax.axis_index("c")
    base = wid * BPW
    pltpu.sync_copy(uid_hbm.at[pl.ds(base, BPW)], uidx_v)
    pltpu.sync_copy(iid_hbm.at[pl.ds(base, BPW)], iidx_v)
    cu = pltpu.async_copy(utab_hbm.at[uidx_v], urows_v, usem)
    ci = pltpu.async_copy(itab_hbm.at[iidx_v], irows_v, isem)
    cu.wait()
    ci.wait()
    pltpu.sync_copy(urows_v, uout_hbm.at[pl.ds(base, BPW)])
    pltpu.sync_copy(irows_v, iout_hbm.at[pl.ds(base, BPW)])


BLK = 2048


def _dense_body(uemb, iemb, uf, itf, wuf, buf_, wif, bif_,
                wufc_t, wufc_b, wifc_t, wifc_b, out):
    ufe = jnp.dot(uf[...], wuf[...], preferred_element_type=jnp.float32) + buf_[...]
    ife = jnp.dot(itf[...], wif[...], preferred_element_type=jnp.float32) + bif_[...]
    fu = jnp.dot(uemb[...], wufc_t[...], preferred_element_type=jnp.float32)
    fu = fu + jnp.dot(ufe, wufc_b[...], preferred_element_type=jnp.float32)
    fi = jnp.dot(iemb[...], wifc_t[...], preferred_element_type=jnp.float32)
    fi = fi + jnp.dot(ife, wifc_b[...], preferred_element_type=jnp.float32)
    fu = jnp.maximum(fu, 0.0)
    fi = jnp.maximum(fi, 0.0)
    out[...] = jnp.sum(fu * fi, axis=1, keepdims=True)


def _dense(uemb, iemb, uf, itf, wuf, buf_, wif, bif_, wufc, wifc):
    grid = (B // BLK,)
    row_spec = lambda w: pl.BlockSpec((BLK, w), lambda i: (i, 0))
    full = lambda a: pl.BlockSpec(a.shape, lambda i: (0,) * a.ndim)
    wufc_t, wufc_b = wufc[:E], wufc[E:]
    wifc_t, wifc_b = wifc[:E], wifc[E:]
    buf2 = buf_.reshape(1, E)
    bif2 = bif_.reshape(1, E)
    return pl.pallas_call(
        _dense_body,
        grid=grid,
        in_specs=[
            row_spec(E), row_spec(E), row_spec(NF), row_spec(NF),
            full(wuf), full(buf2), full(wif), full(bif2),
            full(wufc_t), full(wufc_b), full(wifc_t), full(wifc_b),
        ],
        out_specs=pl.BlockSpec((BLK, 1), lambda i: (i, 0)),
        out_shape=jax.ShapeDtypeStruct((B, 1), jnp.float32),
    )(uemb, iemb, uf, itf, wuf, buf2, wif, bif2,
      wufc_t, wufc_b, wifc_t, wifc_b)


def kernel(user_id, target_item_id, history_item_id, history_len,
           user_features, item_features, user_table, item_table,
           W_uf, b_uf, W_if, b_if, W_ufc, W_ifc):
    uid = user_id.reshape(B).astype(jnp.int32)
    iid = target_item_id.reshape(B).astype(jnp.int32)
    uemb, iemb = _sc_gather2(uid, iid, user_table, item_table)
    return _dense(uemb, iemb, user_features, item_features,
                  W_uf, b_uf, W_if, b_if, W_ufc, W_ifc)
